# baseline (speedup 1.0000x reference)
import jax
import jax.numpy as jnp
from jax.experimental import pallas as pl
from jax.experimental.pallas import tpu as pltpu

_K = 32
_R = 8
_N = 32768
_G = _N // 128


def _cumsum_lanes(a):
    s = 1
    while s < a.shape[-1]:
        pad = jnp.zeros(a.shape[:-1] + (s,), a.dtype)
        a = a + jnp.concatenate([pad, a[..., :-s]], axis=-1)
        s *= 2
    return a


def _cumsum_groups(a):
    s = 1
    while s < a.shape[1]:
        pad = jnp.zeros((a.shape[0], s, a.shape[2]), a.dtype)
        a = a + jnp.concatenate([pad, a[:, :-s, :]], axis=1)
        s *= 2
    return a


def _body(x_ref, o_ref):
    xr = x_ref[...].reshape(_R, _G, 128)

    m = jnp.max(xr, axis=1)

    def ext(_, carry):
        m_cur, _tau = carry
        g = jnp.max(m_cur, axis=-1, keepdims=True)
        eq = (m_cur == g).astype(jnp.int32)
        first = (eq > 0) & (_cumsum_lanes(eq) == 1)
        return jnp.where(first, -jnp.inf, m_cur), g

    _, tau0 = jax.lax.fori_loop(
        0, _K, ext, (m, jnp.zeros((_R, 1), jnp.float32))
    )

    def cnt_gt(t):
        b = (xr > t[:, :, None]).astype(jnp.int32)
        return jnp.sum(jnp.sum(b, axis=1), axis=-1, keepdims=True)

    c0 = cnt_gt(tau0)

    def cond(carry):
        _t, c = carry
        return jnp.any(c >= _K)

    def body(carry):
        tau, c = carry
        masked = jnp.where(xr > tau[:, :, None], xr, jnp.inf)
        nxt = jnp.min(jnp.min(masked, axis=1), axis=-1, keepdims=True)
        newtau = jnp.where(c >= _K, nxt, tau)
        return newtau, cnt_gt(newtau)

    tau, c = jax.lax.while_loop(cond, body, (tau0, c0))
    r = _K - c

    tb = tau[:, :, None]
    c_eq = jnp.sum(
        jnp.sum((xr == tb).astype(jnp.int32), axis=1), axis=-1, keepdims=True
    )
    simple = jnp.all(c_eq <= r)

    @pl.when(simple)
    def _():
        o_ref[...] = jnp.where(xr >= tb, xr, 0.0).reshape(_R, _N)

    @pl.when(jnp.logical_not(simple))
    def _():
        eq = xr == tb
        eqi = eq.astype(jnp.int32)
        within = _cumsum_lanes(eqi)
        grp = within[:, :, 127:128]
        gpref = _cumsum_groups(grp) - grp
        prefix = within - eqi + gpref
        keep = eq & (prefix < r[:, :, None])
        mask = (xr > tb) | keep
        o_ref[...] = jnp.where(mask, xr, 0.0).reshape(_R, _N)


@jax.jit
def kernel(x):
    grid = x.shape[0] // _R
    return pl.pallas_call(
        _body,
        grid=(grid,),
        in_specs=[pl.BlockSpec((_R, _N), lambda i: (i, 0))],
        out_specs=pl.BlockSpec((_R, _N), lambda i: (i, 0)),
        out_shape=jax.ShapeDtypeStruct(x.shape, x.dtype),
        compiler_params=pltpu.CompilerParams(
            dimension_semantics=("parallel",)
        ),
    )(x)


# speedup vs baseline: 2.6595x; 2.6595x over previous
import jax
import jax.numpy as jnp
from jax.experimental import pallas as pl
from jax.experimental.pallas import tpu as pltpu

_K = 32
_R = 16
_N = 32768
_NS = _N // 128
_ACC = 8


def _cumsum_lanes(a):
    s = 1
    while s < a.shape[-1]:
        pad = jnp.zeros(a.shape[:-1] + (s,), a.dtype)
        a = a + jnp.concatenate([pad, a[..., :-s]], axis=-1)
        s *= 2
    return a


def _body(x_ref, o_ref):
    neg = jnp.float32(-jnp.inf)
    pos = jnp.float32(jnp.inf)

    def slices():
        for v in range(_NS):
            yield v, x_ref[:, 128 * v:128 * (v + 1)]

    ms = [jnp.full((_R, 128), neg) for _ in range(_ACC)]
    m2s = [jnp.full((_R, 128), neg) for _ in range(_ACC)]
    for v, xv in slices():
        a = v % _ACC
        m2s[a] = jnp.maximum(m2s[a], jnp.minimum(ms[a], xv))
        ms[a] = jnp.maximum(ms[a], xv)
    step = _ACC
    while step > 1:
        half = step // 2
        for a in range(half):
            b = a + half
            m2s[a] = jnp.maximum(jnp.minimum(ms[a], ms[b]),
                                 jnp.maximum(m2s[a], m2s[b]))
            ms[a] = jnp.maximum(ms[a], ms[b])
        step = half
    cand = jnp.concatenate([ms[0], m2s[0]], axis=-1)

    def ext(_, carry):
        cand_cur, _tau = carry
        g = jnp.max(cand_cur, axis=-1, keepdims=True)
        return jnp.where(cand_cur == g, neg, cand_cur), g

    _, tau0 = jax.lax.fori_loop(
        0, _K, ext, (cand, jnp.zeros((_R, 1), jnp.float32))
    )

    def cnt_gt(t):
        accs = [jnp.zeros((_R, 128), jnp.int32) for _ in range(_ACC)]
        for v, xv in slices():
            a = v % _ACC
            accs[a] = accs[a] + (xv > t).astype(jnp.int32)
        tot = accs[0]
        for a in range(1, _ACC):
            tot = tot + accs[a]
        return jnp.sum(tot, axis=-1, keepdims=True)

    c0 = cnt_gt(tau0)

    def cond(carry):
        _t, c = carry
        return jnp.any(c >= _K)

    def body(carry):
        tau, c = carry
        mns = [jnp.full((_R, 128), pos) for _ in range(_ACC)]
        for v, xv in slices():
            a = v % _ACC
            mns[a] = jnp.minimum(mns[a], jnp.where(xv > tau, xv, pos))
        mn = mns[0]
        for a in range(1, _ACC):
            mn = jnp.minimum(mn, mns[a])
        nxt = jnp.min(mn, axis=-1, keepdims=True)
        newtau = jnp.where(c >= _K, nxt, tau)
        return newtau, cnt_gt(newtau)

    tau, c = jax.lax.while_loop(cond, body, (tau0, c0))
    r = _K - c

    eqs = [jnp.zeros((_R, 128), jnp.int32) for _ in range(_ACC)]
    for v, xv in slices():
        a = v % _ACC
        o_ref[:, 128 * v:128 * (v + 1)] = jnp.where(xv >= tau, xv, 0.0)
        eqs[a] = eqs[a] + (xv == tau).astype(jnp.int32)
    eqt = eqs[0]
    for a in range(1, _ACC):
        eqt = eqt + eqs[a]
    c_eq = jnp.sum(eqt, axis=-1, keepdims=True)

    @pl.when(jnp.logical_not(jnp.all(c_eq <= r)))
    def _():
        base = jnp.zeros((_R, 1), jnp.int32)
        for v, xv in slices():
            eqi = (xv == tau).astype(jnp.int32)
            pref = _cumsum_lanes(eqi) - eqi + base
            keep = (xv > tau) | ((eqi > 0) & (pref < r))
            o_ref[:, 128 * v:128 * (v + 1)] = jnp.where(keep, xv, 0.0)
            base = base + jnp.sum(eqi, axis=-1, keepdims=True)


@jax.jit
def kernel(x):
    grid = x.shape[0] // _R
    return pl.pallas_call(
        _body,
        grid=(grid,),
        in_specs=[pl.BlockSpec((_R, _N), lambda i: (i, 0))],
        out_specs=pl.BlockSpec((_R, _N), lambda i: (i, 0)),
        out_shape=jax.ShapeDtypeStruct(x.shape, x.dtype),
        compiler_params=pltpu.CompilerParams(
            dimension_semantics=("parallel",)
        ),
    )(x)


# speedup vs baseline: 2.8476x; 1.0707x over previous
import jax
import jax.numpy as jnp
from jax.experimental import pallas as pl
from jax.experimental.pallas import tpu as pltpu

_K = 32
_R = 16
_N = 32768
_NS = _N // 128
_ACC = 8


def _cumsum_lanes(a):
    s = 1
    while s < a.shape[-1]:
        pad = jnp.zeros(a.shape[:-1] + (s,), a.dtype)
        a = a + jnp.concatenate([pad, a[..., :-s]], axis=-1)
        s *= 2
    return a


def _body(x_ref, o_ref):
    neg = jnp.float32(-jnp.inf)
    pos = jnp.float32(jnp.inf)

    def slices():
        for v in range(_NS):
            yield v, x_ref[:, 128 * v:128 * (v + 1)]

    ms = [jnp.full((_R, 128), neg) for _ in range(_ACC)]
    m2s = [jnp.full((_R, 128), neg) for _ in range(_ACC)]
    for v, xv in slices():
        a = v % _ACC
        m2s[a] = jnp.maximum(m2s[a], jnp.minimum(ms[a], xv))
        ms[a] = jnp.maximum(ms[a], xv)
    step = _ACC
    while step > 1:
        half = step // 2
        for a in range(half):
            b = a + half
            m2s[a] = jnp.maximum(jnp.minimum(ms[a], ms[b]),
                                 jnp.maximum(m2s[a], m2s[b]))
            ms[a] = jnp.maximum(ms[a], ms[b])
        step = half
    cand = jnp.concatenate([ms[0], m2s[0]], axis=-1)
    ct = cand.T

    def ext(_, carry):
        ct_cur, _tau = carry
        g = jnp.max(ct_cur, axis=0, keepdims=True)
        return jnp.where(ct_cur == g, neg, ct_cur), g

    _, tau0t = jax.lax.fori_loop(
        0, _K, ext, (ct, jnp.zeros((1, _R), jnp.float32))
    )
    tau0 = tau0t.T

    def probe(t):
        cnts = [jnp.zeros((_R, 128), jnp.int32) for _ in range(_ACC)]
        mns = [jnp.full((_R, 128), pos) for _ in range(_ACC)]
        for v, xv in slices():
            a = v % _ACC
            gt = xv > t
            cnts[a] = cnts[a] + gt.astype(jnp.int32)
            mns[a] = jnp.minimum(mns[a], jnp.where(gt, xv, pos))
        cnt, mn = cnts[0], mns[0]
        for a in range(1, _ACC):
            cnt = cnt + cnts[a]
            mn = jnp.minimum(mn, mns[a])
        return (jnp.sum(cnt, axis=-1, keepdims=True),
                jnp.min(mn, axis=-1, keepdims=True))

    c0, nxt0 = probe(tau0)

    def cond(carry):
        _t, c, _n = carry
        return jnp.any(c >= _K)

    def body(carry):
        tau, c, nxt = carry
        newtau = jnp.where(c >= _K, nxt, tau)
        newc, newnxt = probe(newtau)
        return newtau, newc, newnxt

    tau, c, _ = jax.lax.while_loop(cond, body, (tau0, c0, nxt0))
    r = _K - c

    eqs = [jnp.zeros((_R, 128), jnp.int32) for _ in range(_ACC)]
    for v, xv in slices():
        a = v % _ACC
        o_ref[:, 128 * v:128 * (v + 1)] = jnp.where(xv >= tau, xv, 0.0)
        eqs[a] = eqs[a] + (xv == tau).astype(jnp.int32)
    eqt = eqs[0]
    for a in range(1, _ACC):
        eqt = eqt + eqs[a]
    c_eq = jnp.sum(eqt, axis=-1, keepdims=True)

    @pl.when(jnp.logical_not(jnp.all(c_eq <= r)))
    def _():
        base = jnp.zeros((_R, 1), jnp.int32)
        for v, xv in slices():
            eqi = (xv == tau).astype(jnp.int32)
            pref = _cumsum_lanes(eqi) - eqi + base
            keep = (xv > tau) | ((eqi > 0) & (pref < r))
            o_ref[:, 128 * v:128 * (v + 1)] = jnp.where(keep, xv, 0.0)
            base = base + jnp.sum(eqi, axis=-1, keepdims=True)


@jax.jit
def kernel(x):
    grid = x.shape[0] // _R
    return pl.pallas_call(
        _body,
        grid=(grid,),
        in_specs=[pl.BlockSpec((_R, _N), lambda i: (i, 0))],
        out_specs=pl.BlockSpec((_R, _N), lambda i: (i, 0)),
        out_shape=jax.ShapeDtypeStruct(x.shape, x.dtype),
        compiler_params=pltpu.CompilerParams(
            dimension_semantics=("parallel",)
        ),
    )(x)
